# baseline (speedup 1.0000x reference)
import jax
import jax.numpy as jnp
from jax import lax
from jax.experimental import pallas as pl
from jax.experimental.pallas import tpu as pltpu
from jax.experimental.pallas import tpu_sc as plsc

NUM_EMB = 1000000
D = 64
R = 8
SCALE = 16 / 8

NC = 2
NS = 16
NW = NC * NS
L = 16

B_TOTAL = 4096 * 50
B_W = B_TOTAL // NW
GROUP = 128
CHUNK = 640
G_PER_CHUNK = CHUNK // GROUP
N_CHUNKS = B_W // CHUNK


def _sc_body(x_hbm, a_hbm, base_hbm, lora_hbm, out_hbm,
             idx_v, rows_v, lrows_v, lflat_v, lsh_v, a_v, sem):
    sid = lax.axis_index("s")
    wid = sid * NC + lax.axis_index("c")
    wbase = wid * B_W

    pltpu.sync_copy(x_hbm.at[pl.ds(wbase, B_W)], idx_v)
    pltpu.sync_copy(a_hbm, a_v)

    a_regs = [[a_v[r, pl.ds(dv * L, L)] for dv in range(4)]
              for r in range(R)]

    for k in range(N_CHUNKS):
        cbase = k * CHUNK
        copies = []
        for g in range(G_PER_CHUNK):
            isl = idx_v.at[pl.ds(cbase + g * GROUP, GROUP)]
            copies.append(pltpu.async_copy(
                base_hbm.at[isl], rows_v.at[pl.ds(g * GROUP, GROUP)], sem))
            copies.append(pltpu.async_copy(
                lora_hbm.at[isl], lrows_v.at[pl.ds(g * GROUP, GROUP)], sem))
        for c in copies:
            c.wait()

        pltpu.sync_copy(lrows_v, lsh_v.at[sid])
        pltpu.sync_copy(lsh_v.at[sid],
                        lflat_v.at[pl.ds(0, CHUNK), pl.ds(0, R)])

        def row_body(i, carry):
            bv = lflat_v[i, :]
            for dv in range(4):
                acc = rows_v[i, pl.ds(dv * L, L)]
                for r in range(R):
                    acc = acc + bv[r] * a_regs[r][dv]
                rows_v[i, pl.ds(dv * L, L)] = acc
            return carry

        lax.fori_loop(0, CHUNK, row_body, 0)

        pltpu.sync_copy(rows_v, out_hbm.at[pl.ds(wbase + cbase, CHUNK)])


TMID = 1000
TLANE = 1000
TSUB = 8


def _tp_body(bt_ref, ob_ref):
    half = TLANE * D // 128
    for p in range(TSUB):
        vt = bt_ref[:, p, :].T
        v3 = vt.reshape(half, 2, D)
        ob_ref[pl.ds(p * half, half), 0:D] = v3[:, 0, :]
        ob_ref[pl.ds(p * half, half), D:2 * D] = v3[:, 1, :]


def _to_row_major(base_t3):
    rows_per_step = TSUB * TLANE * D // 128
    ob = pl.pallas_call(
        _tp_body,
        grid=(TMID // TSUB,),
        in_specs=[pl.BlockSpec((D, TSUB, TLANE), lambda i: (0, i, 0))],
        out_specs=pl.BlockSpec((rows_per_step, 128), lambda i: (i, 0)),
        out_shape=jax.ShapeDtypeStruct((NUM_EMB * D // 128, 128),
                                       jnp.float32),
    )(base_t3)
    return ob.reshape(NUM_EMB, D)


def kernel(x, base_table, lora_A, lora_B):
    xf = x.reshape(-1)
    a_scaled = lora_A * SCALE
    base_rm = _to_row_major(base_table.T.reshape(D, TMID, TLANE))

    mesh = plsc.VectorSubcoreMesh(core_axis_name="c", subcore_axis_name="s",
                                  num_cores=NC, num_subcores=NS)
    out = pl.kernel(
        _sc_body,
        out_type=jax.ShapeDtypeStruct((B_TOTAL, D), jnp.float32),
        mesh=mesh,
        compiler_params=pltpu.CompilerParams(use_tc_tiling_on_sc=False),
        scratch_types=[
            pltpu.VMEM((B_W,), jnp.int32),
            pltpu.VMEM((CHUNK, D), jnp.float32),
            pltpu.VMEM((CHUNK, R), jnp.float32),
            pltpu.VMEM((CHUNK, 2 * R), jnp.float32),
            pltpu.VMEM_SHARED((NS, CHUNK, R), jnp.float32),
            pltpu.VMEM((R, D), jnp.float32),
            pltpu.SemaphoreType.DMA,
        ],
    )(xf, a_scaled, base_rm, lora_B)
    return out.reshape(x.shape[0], x.shape[1], D)


# speedup vs baseline: 1.0941x; 1.0941x over previous
import jax
import jax.numpy as jnp
from jax import lax
from jax.experimental import pallas as pl
from jax.experimental.pallas import tpu as pltpu
from jax.experimental.pallas import tpu_sc as plsc

NUM_EMB = 1000000
D = 64
R = 8
SCALE = 16 / 8

NC = 2
NS = 16
NW = NC * NS
L = 16

B_TOTAL = 4096 * 50
B_W = B_TOTAL // NW
GROUP = 128
CHUNK = 256
G_PER_CHUNK = CHUNK // GROUP
N_CHUNKS = B_W // CHUNK
UNROLL = 16


def _sc_body(x_hbm, a_hbm, base_hbm, lora_hbm, out_hbm,
             idx_v, ib_v, il_v, qo_v, mo_v, browg_v, lrowg_v, out_v, a_v,
             sem):
    sid = lax.axis_index("s")
    wid = sid * NC + lax.axis_index("c")
    wbase = pl.multiple_of(wid * B_W, B_W)
    obase = pl.multiple_of(wid * (B_W // 2), B_W // 2)

    pltpu.sync_copy(x_hbm.at[pl.ds(wbase, B_W)], idx_v)
    pltpu.sync_copy(a_hbm, a_v)

    def idx_body(t, carry):
        v = idx_v[pl.ds(t * L, L)]
        ib_v[pl.ds(t * L, L)] = v >> 1
        il_v[pl.ds(t * L, L)] = v >> 4
        qo_v[pl.ds(t * L, L)] = (v & 1) * D
        mo_v[pl.ds(t * L, L)] = ((v >> 1) & 7) * L
        return carry

    lax.fori_loop(0, B_W // L, idx_body, 0)

    a_regs = [[a_v[r, pl.ds(dv * L, L)] for dv in range(4)]
              for r in range(R)]
    lane8 = lax.iota(jnp.int32, L) & 7

    def chunk_body(k, carry):
        cbase = pl.multiple_of(k * CHUNK, CHUNK)
        copies = []
        for g in range(G_PER_CHUNK):
            copies.append(pltpu.async_copy(
                base_hbm.at[ib_v.at[pl.ds(cbase + g * GROUP, GROUP)]],
                browg_v.at[pl.ds(g * GROUP, GROUP)], sem))
            copies.append(pltpu.async_copy(
                lora_hbm.at[il_v.at[pl.ds(cbase + g * GROUP, GROUP)]],
                lrowg_v.at[pl.ds(g * GROUP, GROUP)], sem))
        for c in copies:
            c.wait()

        def blk_body(t, carry2):
            qo16 = qo_v[pl.ds(cbase + t * UNROLL, UNROLL)]
            mo16 = mo_v[pl.ds(cbase + t * UNROLL, UNROLL)]
            for u in range(UNROLL):
                i = t * UNROLL + u
                qo = qo16[u]
                bslot = lrowg_v[i, pl.ds(mo16[u], L)]
                bv = bslot.at[lane8 + (qo >> 3)].get(
                    mode="promise_in_bounds")
                for dv in range(4):
                    acc = browg_v[i, pl.ds(qo + dv * L, L)]
                    for r in range(R):
                        acc = acc + bv[r] * a_regs[r][dv]
                    out_v[t * (UNROLL // 2) + u // 2,
                          pl.ds((u & 1) * D + dv * L, L)] = acc
            return carry2

        lax.fori_loop(0, CHUNK // UNROLL, blk_body, 0)

        pltpu.sync_copy(
            out_v,
            out_hbm.at[pl.ds(obase + pl.multiple_of(k * (CHUNK // 2),
                                                    CHUNK // 2),
                             CHUNK // 2)])
        return carry

    lax.fori_loop(0, N_CHUNKS, chunk_body, 0)


def kernel(x, base_table, lora_A, lora_B):
    xf = x.reshape(-1)
    a_scaled = lora_A * SCALE
    base128 = base_table.reshape(NUM_EMB * D // 128, 128)
    lora128 = lora_B.reshape(NUM_EMB * R // 128, 128)

    mesh = plsc.VectorSubcoreMesh(core_axis_name="c", subcore_axis_name="s",
                                  num_cores=NC, num_subcores=NS)
    out = pl.kernel(
        _sc_body,
        out_type=jax.ShapeDtypeStruct((B_TOTAL * D // 128, 128),
                                      jnp.float32),
        mesh=mesh,
        compiler_params=pltpu.CompilerParams(use_tc_tiling_on_sc=True),
        scratch_types=[
            pltpu.VMEM((B_W,), jnp.int32),
            pltpu.VMEM((B_W,), jnp.int32),
            pltpu.VMEM((B_W,), jnp.int32),
            pltpu.VMEM((B_W,), jnp.int32),
            pltpu.VMEM((B_W,), jnp.int32),
            pltpu.VMEM((CHUNK, 128), jnp.float32),
            pltpu.VMEM((CHUNK, 128), jnp.float32),
            pltpu.VMEM((CHUNK // 2, 128), jnp.float32),
            pltpu.VMEM((R, D), jnp.float32),
            pltpu.SemaphoreType.DMA,
        ],
    )(xf, a_scaled, base128, lora128)
    return out.reshape(x.shape[0], x.shape[1], D)


# speedup vs baseline: 1.2090x; 1.1050x over previous
import jax
import jax.numpy as jnp
from jax import lax
from jax.experimental import pallas as pl
from jax.experimental.pallas import tpu as pltpu
from jax.experimental.pallas import tpu_sc as plsc

NUM_EMB = 1000000
D = 64
R = 8
SCALE = 16 / 8

NC = 2
NS = 16
NW = NC * NS
L = 16

B_TOTAL = 4096 * 50
B_W = B_TOTAL // NW
GROUP = 128
CHUNK = 256
G_PER_CHUNK = CHUNK // GROUP
N_CHUNKS = B_W // CHUNK
UNROLL = 16


def _sc_body(x_hbm, a_hbm, base_hbm, lora_hbm, out_hbm,
             idx_v, ib_v, il_v, qo_v, mo_v, browg_v, lrowg_v, out_v, a_v,
             sem):
    sid = lax.axis_index("s")
    wid = sid * NC + lax.axis_index("c")
    wbase = pl.multiple_of(wid * B_W, B_W)
    obase = pl.multiple_of(wid * (B_W // 2), B_W // 2)

    pltpu.sync_copy(x_hbm.at[pl.ds(wbase, B_W)], idx_v)
    pltpu.sync_copy(a_hbm, a_v)

    def idx_body(t, carry):
        v = idx_v[pl.ds(t * L, L)]
        ib_v[pl.ds(t * L, L)] = v >> 1
        il_v[pl.ds(t * L, L)] = v >> 4
        qo_v[pl.ds(t * L, L)] = (v & 1) * D
        mo_v[pl.ds(t * L, L)] = ((v >> 1) & 7) * L
        return carry

    lax.fori_loop(0, B_W // L, idx_body, 0)

    a_regs = [[a_v[r, pl.ds(dv * L, L)] for dv in range(4)]
              for r in range(R)]
    lane8 = lax.iota(jnp.int32, L) & 7

    def chunk_body(k, carry):
        cbase = pl.multiple_of(k * CHUNK, CHUNK)
        copies = []
        for g in range(G_PER_CHUNK):
            copies.append(pltpu.async_copy(
                base_hbm.at[ib_v.at[pl.ds(cbase + g * GROUP, GROUP)]],
                browg_v.at[pl.ds(g * GROUP, GROUP)], sem))
            copies.append(pltpu.async_copy(
                lora_hbm.at[il_v.at[pl.ds(cbase + g * GROUP, GROUP)]],
                lrowg_v.at[pl.ds(g * GROUP, GROUP)], sem))
        for c in copies:
            c.wait()

        def blk_body(t, carry2):
            qo16 = qo_v[pl.ds(cbase + t * UNROLL, UNROLL)]
            mo16 = mo_v[pl.ds(cbase + t * UNROLL, UNROLL)]
            for u in range(UNROLL):
                i = t * UNROLL + u
                qo = qo16[u]
                bslot = lrowg_v[i, pl.ds(mo16[u], L)]
                bv = bslot.at[lane8 + (qo >> 3)].get(
                    mode="promise_in_bounds")
                for dv in range(4):
                    acc = browg_v[i, pl.ds(qo + dv * L, L)]
                    for r in range(R):
                        acc = acc + bv[r] * a_regs[r][dv]
                    out_v[t * (UNROLL // 2) + u // 2,
                          pl.ds((u & 1) * D + dv * L, L)] = acc
            return carry2

        lax.fori_loop(0, CHUNK // UNROLL, blk_body, 0)

        pltpu.sync_copy(
            out_v,
            out_hbm.at[pl.ds(obase + pl.multiple_of(k * (CHUNK // 2),
                                                    CHUNK // 2),
                             CHUNK // 2)])
        return carry

    lax.fori_loop(0, N_CHUNKS, chunk_body, 0)


TBLK = 4096


def _tp_body(bt_ref, ob_ref):
    half = TBLK // 2
    vt = bt_ref[...].T
    v3 = vt.reshape(half, 2, D)
    ob_ref[pl.ds(0, half), 0:D] = v3[:, 0, :]
    ob_ref[pl.ds(0, half), D:2 * D] = v3[:, 1, :]


def _to_row_major(base_t):
    grid = (NUM_EMB + TBLK - 1) // TBLK
    return pl.pallas_call(
        _tp_body,
        grid=(grid,),
        in_specs=[pl.BlockSpec((D, TBLK), lambda i: (0, i))],
        out_specs=pl.BlockSpec((TBLK // 2, 128), lambda i: (i, 0)),
        out_shape=jax.ShapeDtypeStruct((NUM_EMB * D // 128, 128),
                                       jnp.float32),
    )(base_t)


def kernel(x, base_table, lora_A, lora_B):
    xf = x.reshape(-1)
    a_scaled = lora_A * SCALE
    base128 = _to_row_major(base_table.T)
    lora128 = lora_B.reshape(NUM_EMB * R // 128, 128)

    mesh = plsc.VectorSubcoreMesh(core_axis_name="c", subcore_axis_name="s",
                                  num_cores=NC, num_subcores=NS)
    out = pl.kernel(
        _sc_body,
        out_type=jax.ShapeDtypeStruct((B_TOTAL * D // 128, 128),
                                      jnp.float32),
        mesh=mesh,
        compiler_params=pltpu.CompilerParams(use_tc_tiling_on_sc=True),
        scratch_types=[
            pltpu.VMEM((B_W,), jnp.int32),
            pltpu.VMEM((B_W,), jnp.int32),
            pltpu.VMEM((B_W,), jnp.int32),
            pltpu.VMEM((B_W,), jnp.int32),
            pltpu.VMEM((B_W,), jnp.int32),
            pltpu.VMEM((CHUNK, 128), jnp.float32),
            pltpu.VMEM((CHUNK, 128), jnp.float32),
            pltpu.VMEM((CHUNK // 2, 128), jnp.float32),
            pltpu.VMEM((R, D), jnp.float32),
            pltpu.SemaphoreType.DMA,
        ],
    )(xf, a_scaled, base128, lora128)
    return out.reshape(x.shape[0], x.shape[1], D)


# speedup vs baseline: 1.2830x; 1.0611x over previous
import jax
import jax.numpy as jnp
from jax import lax
from jax.experimental import pallas as pl
from jax.experimental.pallas import tpu as pltpu
from jax.experimental.pallas import tpu_sc as plsc

NUM_EMB = 1000000
D = 64
R = 8
SCALE = 16 / 8

NC = 2
NS = 16
NW = NC * NS
L = 16

B_TOTAL = 4096 * 50
B_W = B_TOTAL // NW
GROUP = 128
CHUNK = 128
G_PER_CHUNK = CHUNK // GROUP
N_CHUNKS = B_W // CHUNK
UNROLL = 16


def _sc_body(x_hbm, a_hbm, base_hbm, lora_hbm, out_hbm,
             idx_v, ib_v, il_v, qo_v, mo_v, browg_v, lrowg_v, out_v, a_v,
             sem):
    sid = lax.axis_index("s")
    wid = sid * NC + lax.axis_index("c")
    wbase = pl.multiple_of(wid * B_W, B_W)
    obase = pl.multiple_of(wid * (B_W // 2), B_W // 2)

    pltpu.sync_copy(x_hbm.at[pl.ds(wbase, B_W)], idx_v)
    pltpu.sync_copy(a_hbm, a_v)

    def idx_body(t, carry):
        v = idx_v[pl.ds(t * L, L)]
        ib_v[pl.ds(t * L, L)] = v >> 1
        il_v[pl.ds(t * L, L)] = v >> 4
        qo_v[pl.ds(t * L, L)] = (v & 1) * D
        mo_v[pl.ds(t * L, L)] = ((v >> 1) & 7) * L
        return carry

    lax.fori_loop(0, B_W // L, idx_body, 0)

    a_regs = [[a_v[r, pl.ds(dv * L, L)] for dv in range(4)]
              for r in range(R)]
    lane8 = lax.iota(jnp.int32, L) & 7

    def mk_copies(k, slot):
        cbase = pl.multiple_of(k * CHUNK, CHUNK)
        return (
            pltpu.make_async_copy(
                base_hbm.at[ib_v.at[pl.ds(cbase, CHUNK)]],
                browg_v.at[slot], sem.at[slot]),
            pltpu.make_async_copy(
                lora_hbm.at[il_v.at[pl.ds(cbase, CHUNK)]],
                lrowg_v.at[slot], sem.at[slot]),
        )

    for c in mk_copies(0, 0):
        c.start()

    def chunk_body(k, carry):
        slot = k & 1
        cbase = pl.multiple_of(k * CHUNK, CHUNK)

        @pl.when(k + 1 < N_CHUNKS)
        def _fire_next():
            for c in mk_copies(k + 1, (k + 1) & 1):
                c.start()

        for c in mk_copies(k, slot):
            c.wait()

        def blk_body(t, carry2):
            qo16 = qo_v[pl.ds(cbase + t * UNROLL, UNROLL)]
            mo16 = mo_v[pl.ds(cbase + t * UNROLL, UNROLL)]
            for u in range(UNROLL):
                i = t * UNROLL + u
                qo = qo16[u]
                bslot = lrowg_v[slot, i, pl.ds(mo16[u], L)]
                bv = bslot.at[lane8 + (qo >> 3)].get(
                    mode="promise_in_bounds")
                for dv in range(4):
                    acc = browg_v[slot, i, pl.ds(qo + dv * L, L)]
                    for r in range(R):
                        acc = acc + bv[r] * a_regs[r][dv]
                    out_v[t * (UNROLL // 2) + u // 2,
                          pl.ds((u & 1) * D + dv * L, L)] = acc
            return carry2

        lax.fori_loop(0, CHUNK // UNROLL, blk_body, 0)

        pltpu.sync_copy(
            out_v,
            out_hbm.at[pl.ds(obase + pl.multiple_of(k * (CHUNK // 2),
                                                    CHUNK // 2),
                             CHUNK // 2)])
        return carry

    lax.fori_loop(0, N_CHUNKS, chunk_body, 0)


TBLK = 4096


def _tp_body(bt_ref, ob_ref):
    half = TBLK // 2
    vt = bt_ref[...].T
    v3 = vt.reshape(half, 2, D)
    ob_ref[pl.ds(0, half), 0:D] = v3[:, 0, :]
    ob_ref[pl.ds(0, half), D:2 * D] = v3[:, 1, :]


def _to_row_major(base_t):
    grid = (NUM_EMB + TBLK - 1) // TBLK
    return pl.pallas_call(
        _tp_body,
        grid=(grid,),
        in_specs=[pl.BlockSpec((D, TBLK), lambda i: (0, i))],
        out_specs=pl.BlockSpec((TBLK // 2, 128), lambda i: (i, 0)),
        out_shape=jax.ShapeDtypeStruct((NUM_EMB * D // 128, 128),
                                       jnp.float32),
    )(base_t)


def kernel(x, base_table, lora_A, lora_B):
    xf = x.reshape(-1)
    a_scaled = lora_A * SCALE
    base128 = _to_row_major(base_table.T)
    lora128 = lora_B.reshape(NUM_EMB * R // 128, 128)

    mesh = plsc.VectorSubcoreMesh(core_axis_name="c", subcore_axis_name="s",
                                  num_cores=NC, num_subcores=NS)
    out = pl.kernel(
        _sc_body,
        out_type=jax.ShapeDtypeStruct((B_TOTAL * D // 128, 128),
                                      jnp.float32),
        mesh=mesh,
        compiler_params=pltpu.CompilerParams(use_tc_tiling_on_sc=True),
        scratch_types=[
            pltpu.VMEM((B_W,), jnp.int32),
            pltpu.VMEM((B_W,), jnp.int32),
            pltpu.VMEM((B_W,), jnp.int32),
            pltpu.VMEM((B_W,), jnp.int32),
            pltpu.VMEM((B_W,), jnp.int32),
            pltpu.VMEM((2, CHUNK, 128), jnp.float32),
            pltpu.VMEM((2, CHUNK, 128), jnp.float32),
            pltpu.VMEM((CHUNK // 2, 128), jnp.float32),
            pltpu.VMEM((R, D), jnp.float32),
            pltpu.SemaphoreType.DMA((2,)),
        ],
    )(xf, a_scaled, base128, lora128)
    return out.reshape(x.shape[0], x.shape[1], D)


# speedup vs baseline: 1.4245x; 1.1103x over previous
import jax
import jax.numpy as jnp
from jax import lax
from jax.experimental import pallas as pl
from jax.experimental.pallas import tpu as pltpu
from jax.experimental.pallas import tpu_sc as plsc

NUM_EMB = 1000000
D = 64
R = 8
SCALE = 16 / 8

NC = 2
NS = 16
NW = NC * NS
L = 16

B_TOTAL = 4096 * 50
B_W = B_TOTAL // NW
GROUP = 128
CHUNK = 128
G_PER_CHUNK = CHUNK // GROUP
N_CHUNKS = B_W // CHUNK
UNROLL = 16


def _sc_body(x_hbm, a_hbm, base_hbm, lora_hbm, out_hbm,
             idx_v, ib_v, il_v, qo_v, mo_v, browg_v, lrowg_v, out_v, a_v,
             sem):
    sid = lax.axis_index("s")
    wid = sid * NC + lax.axis_index("c")
    wbase = pl.multiple_of(wid * B_W, B_W)
    obase = pl.multiple_of(wid * (B_W // 2), B_W // 2)

    pltpu.sync_copy(x_hbm.at[pl.ds(wbase, B_W)], idx_v)
    pltpu.sync_copy(a_hbm, a_v)

    def idx_body(t, carry):
        v = idx_v[pl.ds(t * L, L)]
        ib_v[pl.ds(t * L, L)] = v >> 1
        il_v[pl.ds(t * L, L)] = v >> 4
        qo_v[pl.ds(t * L, L)] = (v & 1) * D
        mo_v[pl.ds(t * L, L)] = ((v >> 1) & 7) * L
        return carry

    lax.fori_loop(0, B_W // L, idx_body, 0)

    a_regs = [[a_v[r, pl.ds(dv * L, L)] for dv in range(4)]
              for r in range(R)]
    lane8 = lax.iota(jnp.int32, L) & 7

    def mk_copies(k, slot):
        cbase = pl.multiple_of(k * CHUNK, CHUNK)
        return (
            pltpu.make_async_copy(
                base_hbm.at[ib_v.at[pl.ds(cbase, CHUNK)]],
                browg_v.at[slot], sem.at[slot]),
            pltpu.make_async_copy(
                lora_hbm.at[il_v.at[pl.ds(cbase, CHUNK)]],
                lrowg_v.at[slot], sem.at[slot]),
        )

    for c in mk_copies(0, 0):
        c.start()

    def chunk_body(k, carry):
        slot = k & 1
        cbase = pl.multiple_of(k * CHUNK, CHUNK)

        @pl.when(k + 1 < N_CHUNKS)
        def _fire_next():
            for c in mk_copies(k + 1, (k + 1) & 1):
                c.start()

        for c in mk_copies(k, slot):
            c.wait()

        def blk_body(t, carry2):
            qo16 = qo_v[pl.ds(cbase + t * UNROLL, UNROLL)]
            mo16 = mo_v[pl.ds(cbase + t * UNROLL, UNROLL)]
            for u in range(UNROLL):
                i = t * UNROLL + u
                qo = qo16[u]
                bslot = lrowg_v[slot, i, pl.ds(mo16[u], L)]
                bv = bslot.at[lane8 + (qo >> 3)].get(
                    mode="promise_in_bounds")
                for dv in range(4):
                    acc = browg_v[slot, i, pl.ds(qo + dv * L, L)]
                    for r in range(R):
                        acc = acc + bv[r] * a_regs[r][dv]
                    out_v[t * (UNROLL // 2) + u // 2,
                          pl.ds((u & 1) * D + dv * L, L)] = acc
            return carry2

        lax.fori_loop(0, CHUNK // UNROLL, blk_body, 0)

        pltpu.sync_copy(
            out_v,
            out_hbm.at[pl.ds(obase + pl.multiple_of(k * (CHUNK // 2),
                                                    CHUNK // 2),
                             CHUNK // 2)])
        return carry

    lax.fori_loop(0, N_CHUNKS, chunk_body, 0)


TBLK = 4096


def _tp_body(bt_ref, lt_ref, ob_ref, ol_ref):
    half = TBLK // 2
    vt = bt_ref[...].T
    v3 = vt.reshape(half, 2, D)
    ob_ref[pl.ds(0, half), 0:D] = v3[:, 0, :]
    ob_ref[pl.ds(0, half), D:2 * D] = v3[:, 1, :]

    lt = lt_ref[...].T
    l3 = lt.reshape(TBLK // 16, 16, R)
    for s in range(16):
        ol_ref[pl.ds(0, TBLK // 16), s * R:(s + 1) * R] = l3[:, s, :]


def _to_row_major(base_t, lora_t):
    grid = (NUM_EMB + TBLK - 1) // TBLK
    return pl.pallas_call(
        _tp_body,
        grid=(grid,),
        in_specs=[
            pl.BlockSpec((D, TBLK), lambda i: (0, i)),
            pl.BlockSpec((R, TBLK), lambda i: (0, i)),
        ],
        out_specs=[
            pl.BlockSpec((TBLK // 2, 128), lambda i: (i, 0)),
            pl.BlockSpec((TBLK // 16, 128), lambda i: (i, 0)),
        ],
        out_shape=[
            jax.ShapeDtypeStruct((NUM_EMB * D // 128, 128), jnp.float32),
            jax.ShapeDtypeStruct((NUM_EMB * R // 128, 128), jnp.float32),
        ],
    )(base_t, lora_t)


def kernel(x, base_table, lora_A, lora_B):
    xf = x.reshape(-1)
    a_scaled = lora_A * SCALE
    base128, lora128 = _to_row_major(base_table.T, lora_B.T)

    mesh = plsc.VectorSubcoreMesh(core_axis_name="c", subcore_axis_name="s",
                                  num_cores=NC, num_subcores=NS)
    out = pl.kernel(
        _sc_body,
        out_type=jax.ShapeDtypeStruct((B_TOTAL * D // 128, 128),
                                      jnp.float32),
        mesh=mesh,
        compiler_params=pltpu.CompilerParams(use_tc_tiling_on_sc=True),
        scratch_types=[
            pltpu.VMEM((B_W,), jnp.int32),
            pltpu.VMEM((B_W,), jnp.int32),
            pltpu.VMEM((B_W,), jnp.int32),
            pltpu.VMEM((B_W,), jnp.int32),
            pltpu.VMEM((B_W,), jnp.int32),
            pltpu.VMEM((2, CHUNK, 128), jnp.float32),
            pltpu.VMEM((2, CHUNK, 128), jnp.float32),
            pltpu.VMEM((CHUNK // 2, 128), jnp.float32),
            pltpu.VMEM((R, D), jnp.float32),
            pltpu.SemaphoreType.DMA((2,)),
        ],
    )(xf, a_scaled, base128, lora128)
    return out.reshape(x.shape[0], x.shape[1], D)
